# baseline (speedup 1.0000x reference)
import jax
import jax.numpy as jnp
from jax.experimental import pallas as pl

BLOCK_S = 512


def _add_kernel(x_ref, w_ref, o_ref):
    o_ref[...] = x_ref[...] + w_ref[...]


def kernel(inputs, weight):
    batch, seq_len, dim = inputs.shape
    w = weight[:seq_len]
    grid = (seq_len // BLOCK_S, batch)
    return pl.pallas_call(
        _add_kernel,
        grid=grid,
        in_specs=[
            pl.BlockSpec((1, BLOCK_S, dim), lambda s, b: (b, s, 0)),
            pl.BlockSpec((BLOCK_S, dim), lambda s, b: (s, 0)),
        ],
        out_specs=pl.BlockSpec((1, BLOCK_S, dim), lambda s, b: (b, s, 0)),
        out_shape=jax.ShapeDtypeStruct(inputs.shape, inputs.dtype),
    )(inputs, w)


# speedup vs baseline: 1.1372x; 1.1372x over previous
import jax
import jax.numpy as jnp
from jax.experimental import pallas as pl

BLOCK_S = 256


def _add_kernel(x_ref, w_ref, o_ref):
    o_ref[...] = x_ref[...] + w_ref[...]


def kernel(inputs, weight):
    batch, seq_len, dim = inputs.shape
    w = weight[:seq_len]
    grid = (seq_len // BLOCK_S,)
    return pl.pallas_call(
        _add_kernel,
        grid=grid,
        in_specs=[
            pl.BlockSpec((batch, BLOCK_S, dim), lambda s: (0, s, 0)),
            pl.BlockSpec((BLOCK_S, dim), lambda s: (s, 0)),
        ],
        out_specs=pl.BlockSpec((batch, BLOCK_S, dim), lambda s: (0, s, 0)),
        out_shape=jax.ShapeDtypeStruct(inputs.shape, inputs.dtype),
    )(inputs, w)


# speedup vs baseline: 1.1597x; 1.0198x over previous
import jax
import jax.numpy as jnp
from jax.experimental import pallas as pl

BLOCK_S = 512


def _add_kernel(x_ref, w_ref, o_ref):
    o_ref[...] = x_ref[...] + w_ref[...]


def kernel(inputs, weight):
    batch, seq_len, dim = inputs.shape
    w = weight[:seq_len]
    grid = (seq_len // BLOCK_S,)
    return pl.pallas_call(
        _add_kernel,
        grid=grid,
        in_specs=[
            pl.BlockSpec((batch, BLOCK_S, dim), lambda s: (0, s, 0)),
            pl.BlockSpec((BLOCK_S, dim), lambda s: (s, 0)),
        ],
        out_specs=pl.BlockSpec((batch, BLOCK_S, dim), lambda s: (0, s, 0)),
        out_shape=jax.ShapeDtypeStruct(inputs.shape, inputs.dtype),
    )(inputs, w)
